# baseline (speedup 1.0000x reference)
import math
import functools

import jax
import jax.numpy as jnp
from jax import lax
from jax.experimental import pallas as pl
from jax.experimental.pallas import tpu as pltpu
from jax.experimental.pallas import tpu_sc as plsc

VOCAB = 1000000
EMBED = 64
D_MODEL = 128
B = 4096
L = 200

NC = 2
NS = 16
NW = NC * NS

N = B * L
R = N // NW
CH = 8
ROWS_PER_STREAM = 128
GROUP = CH * ROWS_PER_STREAM
NGROUP = R // GROUP


def _sc_gather(idx3, table):
    mesh = plsc.VectorSubcoreMesh(core_axis_name="c", subcore_axis_name="s")

    @functools.partial(
        pl.kernel,
        out_type=jax.ShapeDtypeStruct((N, EMBED), jnp.float32),
        mesh=mesh,
        scratch_types=[
            pltpu.VMEM((CH, ROWS_PER_STREAM), jnp.int32),
            pltpu.VMEM((GROUP, EMBED), jnp.float32),
            pltpu.SemaphoreType.DMA,
        ],
        compiler_params=pltpu.CompilerParams(use_tc_tiling_on_sc=False),
    )
    def k(idx_hbm, table_hbm, emb_hbm, idx_v, rows_v, sem):
        wid = lax.axis_index("s") * NC + lax.axis_index("c")
        base = wid * R

        def group_body(g, carry):
            pltpu.sync_copy(idx_hbm.at[wid, pl.ds(g * CH, CH)], idx_v)
            descs = [
                pltpu.async_copy(
                    table_hbm.at[idx_v.at[j]],
                    rows_v.at[pl.ds(j * ROWS_PER_STREAM, ROWS_PER_STREAM)],
                    sem,
                )
                for j in range(CH)
            ]
            for d in descs:
                d.wait()
            pltpu.sync_copy(rows_v, emb_hbm.at[pl.ds(base + g * GROUP, GROUP)])
            return carry

        lax.fori_loop(0, NGROUP, group_body, 0)

    return k(idx3, table)


BLK = 2048


def _tc_matmul(emb, Ws, bs):
    def body(emb_ref, w_ref, b_ref, out_ref):
        out_ref[...] = (
            jnp.dot(emb_ref[...], w_ref[...], preferred_element_type=jnp.float32)
            + b_ref[...]
        )

    return pl.pallas_call(
        body,
        grid=(N // BLK,),
        in_specs=[
            pl.BlockSpec((BLK, EMBED), lambda i: (i, 0)),
            pl.BlockSpec((EMBED, D_MODEL), lambda i: (0, 0)),
            pl.BlockSpec((1, D_MODEL), lambda i: (0, 0)),
        ],
        out_specs=pl.BlockSpec((BLK, D_MODEL), lambda i: (i, 0)),
        out_shape=jax.ShapeDtypeStruct((N, D_MODEL), jnp.float32),
    )(emb, Ws, bs)


def kernel(x, table, W, b):
    scale = math.sqrt(D_MODEL)
    idx3 = x.reshape(NW, R // ROWS_PER_STREAM, ROWS_PER_STREAM).astype(jnp.int32)
    emb = _sc_gather(idx3, table)
    out = _tc_matmul(emb, W * scale, (b * scale).reshape(1, D_MODEL))
    return out.reshape(B, L, D_MODEL)
